# baseline (speedup 1.0000x reference)
import functools
import jax
import jax.numpy as jnp
from jax import lax
from jax.experimental import pallas as pl
from jax.experimental.pallas import tpu as pltpu
from jax.experimental.pallas import tpu_sc as plsc

NUM_EXPERTS = 8
HIDDEN = 1024
INTER = 2048
EPS = 1e-6
T = 2048
A = 2 * T
TM = 128
NB = 40
NBTM = NB * TM
NW = 32
APW = A // NW
TPW = T // NW
CH = 32


def _route_body(x_ref, g_ref, rw_ref, tf_ref, pos_ref, gates_ref, bexp_ref):
    xv = x_ref[...]
    ms = jnp.mean(xv * xv, axis=-1, keepdims=True)
    t = xv * lax.rsqrt(ms + EPS) * g_ref[...]
    tf_ref[...] = t
    logits = jnp.dot(t.astype(jnp.bfloat16), rw_ref[...].astype(jnp.bfloat16),
                     preferred_element_type=jnp.float32)
    m = jnp.max(logits, axis=-1, keepdims=True)
    p = jnp.exp(logits - m)
    probs = p / jnp.sum(p, axis=-1, keepdims=True)
    cols = lax.broadcasted_iota(jnp.int32, probs.shape, 1)
    m1 = jnp.max(probs, axis=-1, keepdims=True)
    idx1 = jnp.min(jnp.where(probs == m1, cols, NUM_EXPERTS),
                   axis=-1, keepdims=True)
    probs2 = jnp.where(cols == idx1, -jnp.inf, probs)
    m2 = jnp.max(probs2, axis=-1, keepdims=True)
    idx2 = jnp.min(jnp.where(probs2 == m2, cols, NUM_EXPERTS),
                   axis=-1, keepdims=True)
    denom = m1 + m2
    g0b = jnp.broadcast_to(m1 / denom, (T, 16))
    g1b = jnp.broadcast_to(m2 / denom, (T, 16))
    gates_ref[...] = jnp.concatenate([g0b, g1b], axis=-1)

    o0 = (cols == idx1).astype(jnp.float32)
    o1 = (cols == idx2).astype(jnp.float32)
    occ = o0 + o1
    rt = lax.broadcasted_iota(jnp.int32, (T, T), 0)
    ct = lax.broadcasted_iota(jnp.int32, (T, T), 1)
    ls = (ct < rt).astype(jnp.bfloat16)
    cume = jnp.dot(ls, occ.astype(jnp.bfloat16),
                   preferred_element_type=jnp.float32)
    counts = jnp.sum(occ, axis=0, keepdims=True)
    ci = counts.astype(jnp.int32)
    pci = ((ci + TM - 1) // TM) * TM
    pcf = pci.astype(jnp.float32)
    r8 = lax.broadcasted_iota(jnp.int32, (NUM_EXPERTS, NUM_EXPERTS), 0)
    c8 = lax.broadcasted_iota(jnp.int32, (NUM_EXPERTS, NUM_EXPERTS), 1)
    lt8 = (r8 < c8).astype(jnp.float32)
    off = jnp.dot(pcf, lt8, precision=lax.Precision.HIGHEST,
                  preferred_element_type=jnp.float32)
    base = cume + off
    pos0 = jnp.sum(o0 * base, axis=-1, keepdims=True)
    pos1 = jnp.sum(o1 * base, axis=-1, keepdims=True)
    pos_ref[...] = jnp.concatenate([pos0, pos1], axis=-1).astype(jnp.int32)

    bvals = (lax.broadcasted_iota(jnp.int32, (1, NB), 1) * TM).astype(
        jnp.float32)
    acc = jnp.full((1, NB), -1, dtype=jnp.int32)
    for e in range(NUM_EXPERTS):
        off_e = lax.slice(off, (0, e), (1, e + 1))
        acc = acc + (bvals >= off_e).astype(jnp.int32)
    bexp_ref[...] = acc


def _route(xf, gm, rw):
    return pl.pallas_call(
        _route_body,
        in_specs=[
            pl.BlockSpec((T, HIDDEN), lambda: (0, 0)),
            pl.BlockSpec((1, HIDDEN), lambda: (0, 0)),
            pl.BlockSpec((HIDDEN, NUM_EXPERTS), lambda: (0, 0)),
        ],
        out_specs=[
            pl.BlockSpec((T, HIDDEN), lambda: (0, 0)),
            pl.BlockSpec((T, 2), lambda: (0, 0)),
            pl.BlockSpec((T, 32), lambda: (0, 0)),
            pl.BlockSpec((1, NB), lambda: (0, 0)),
        ],
        out_shape=[
            jax.ShapeDtypeStruct((T, HIDDEN), jnp.float32),
            jax.ShapeDtypeStruct((T, 2), jnp.int32),
            jax.ShapeDtypeStruct((T, 32), jnp.float32),
            jax.ShapeDtypeStruct((1, NB), jnp.int32),
        ],
    )(xf, gm, rw)


def _make_dispatch():
    mesh = plsc.VectorSubcoreMesh(core_axis_name="c", subcore_axis_name="s")

    rpr = APW // 2

    @functools.partial(
        pl.kernel, mesh=mesh,
        out_type=jax.ShapeDtypeStruct((NBTM, HIDDEN), jnp.float32),
        scratch_types=[
            pltpu.VMEM((rpr,), jnp.int32),
            pltpu.VMEM((rpr,), jnp.int32),
            pltpu.VMEM((rpr, HIDDEN), jnp.float32),
            pltpu.SemaphoreType.DMA,
            pltpu.SemaphoreType.DMA,
        ],
    )
    def dispatch(tf_hbm, pos_hbm, xs_hbm, tokv, posv, rows, sg, ss):
        wid = lax.axis_index("s") * 2 + lax.axis_index("c")
        lanes = lax.iota(jnp.int32, 16)
        for r in range(2):
            pltpu.sync_copy(pos_hbm.at[wid, r], posv)
            tok0 = wid * (APW // 2) + r * (rpr // 2)
            for j in range(rpr // 16):
                tokv[pl.ds(j * 16, 16)] = tok0 + ((lanes + j * 16) >> 1)
            pltpu.async_copy(tf_hbm.at[tokv], rows, sg).wait()
            pltpu.async_copy(rows, xs_hbm.at[posv], ss).wait()

    return dispatch


def _mlp_body(be_ref, xs_ref, wg_ref, wl_ref, bgu_ref, wd_ref, bd_ref,
              ys_ref):
    del be_ref
    xb = xs_ref[...].astype(jnp.bfloat16)
    bgu = bgu_ref[0]
    hg = jnp.dot(xb, wg_ref[0], preferred_element_type=jnp.float32) + bgu[0:1]
    hl = jnp.dot(xb, wl_ref[0], preferred_element_type=jnp.float32) + bgu[1:2]
    hg = jnp.minimum(hg, 7.0)
    hl = jnp.clip(hl, -7.0, 7.0)
    act = hg * jax.nn.sigmoid(1.702 * hg) * (hl + 1.0)
    ys_ref[...] = (jnp.dot(act.astype(jnp.bfloat16), wd_ref[0],
                           preferred_element_type=jnp.float32) + bd_ref[0])


def _mlp(bexp, xs, wg, wl, bgu2, wd, bd3):
    grid_spec = pltpu.PrefetchScalarGridSpec(
        num_scalar_prefetch=1,
        grid=(NB,),
        in_specs=[
            pl.BlockSpec((TM, HIDDEN), lambda b, be: (b, 0)),
            pl.BlockSpec((1, HIDDEN, INTER), lambda b, be: (be[b], 0, 0)),
            pl.BlockSpec((1, HIDDEN, INTER), lambda b, be: (be[b], 0, 0)),
            pl.BlockSpec((1, 2, INTER), lambda b, be: (be[b], 0, 0)),
            pl.BlockSpec((1, INTER, HIDDEN), lambda b, be: (be[b], 0, 0)),
            pl.BlockSpec((1, 1, HIDDEN), lambda b, be: (be[b], 0, 0)),
        ],
        out_specs=pl.BlockSpec((TM, HIDDEN), lambda b, be: (b, 0)),
    )
    return pl.pallas_call(
        _mlp_body,
        grid_spec=grid_spec,
        out_shape=jax.ShapeDtypeStruct((NBTM, HIDDEN), jnp.float32),
        compiler_params=pltpu.CompilerParams(
            dimension_semantics=("arbitrary",),
        ),
    )(bexp, xs, wg, wl, bgu2, wd, bd3)


def _make_combine():
    mesh = plsc.VectorSubcoreMesh(core_axis_name="c", subcore_axis_name="s")

    @functools.partial(
        pl.kernel, mesh=mesh,
        out_type=jax.ShapeDtypeStruct((T, HIDDEN), jnp.float32),
        scratch_types=[
            pltpu.VMEM((2 * CH,), jnp.int32),
            pltpu.VMEM((CH, 32), jnp.float32),
            pltpu.VMEM((2 * CH, HIDDEN), jnp.float32),
            pltpu.VMEM((CH, HIDDEN), jnp.float32),
            pltpu.SemaphoreType.DMA,
        ],
    )
    def combine(ys_hbm, pos_hbm, gates_hbm, out_hbm, pv, gm, rows, o, sa):
        wid = lax.axis_index("s") * 2 + lax.axis_index("c")
        for c in range(TPW // CH):
            tok0 = wid * TPW + c * CH
            pltpu.sync_copy(pos_hbm.at[pl.ds(2 * tok0, 2 * CH)], pv)
            pltpu.sync_copy(gates_hbm.at[pl.ds(tok0, CH)], gm)
            pltpu.async_copy(ys_hbm.at[pv], rows, sa).wait()

            def body(j, carry):
                g0 = gm[j, pl.ds(0, 16)]
                g1 = gm[j, pl.ds(16, 16)]
                for u in range(HIDDEN // 16):
                    s = pl.ds(u * 16, 16)
                    o[j, s] = g0 * rows[2 * j, s] + g1 * rows[2 * j + 1, s]
                return carry

            lax.fori_loop(0, CH, body, 0)
            pltpu.sync_copy(o, out_hbm.at[pl.ds(tok0, CH)])

    return combine


def kernel(x, norm_g, router_w, W_gu, b_gu, W_d, b_d):
    b, s, h = x.shape
    xf = x.reshape(T, h)
    gm = norm_g.reshape(1, h)
    wgu4 = W_gu.reshape(NUM_EXPERTS, h, INTER, 2)
    W_g = wgu4[..., 0].astype(jnp.bfloat16)
    W_l = wgu4[..., 1].astype(jnp.bfloat16)
    bgu2 = jnp.swapaxes(b_gu.reshape(NUM_EXPERTS, INTER, 2), 1, 2)
    W_d_bf = W_d.astype(jnp.bfloat16)
    bd3 = b_d.reshape(NUM_EXPERTS, 1, h)

    tf, pos2, gates32, bexp = _route(xf, gm, router_w)

    dispatch = _make_dispatch()
    xs = dispatch(tf, pos2.reshape(NW, 2, APW // 2))

    ys = _mlp(bexp.reshape(NB), xs, W_g, W_l, bgu2, W_d_bf, bd3)

    combine = _make_combine()
    out = combine(ys, pos2.reshape(A), gates32)
    return out.reshape(b, s, h)


# speedup vs baseline: 1.0114x; 1.0114x over previous
import functools
import jax
import jax.numpy as jnp
from jax import lax
from jax.experimental import pallas as pl
from jax.experimental.pallas import tpu as pltpu
from jax.experimental.pallas import tpu_sc as plsc

NUM_EXPERTS = 8
HIDDEN = 1024
INTER = 2048
EPS = 1e-6
T = 2048
A = 2 * T
TM = 256
NB = 24
NBTM = NB * TM
NW = 32
APW = A // NW
TPW = T // NW
CH = 16


def _route_body(x_ref, g_ref, rw_ref, tf_ref, pos_ref, gates_ref, bexp_ref):
    xv = x_ref[...]
    ms = jnp.mean(xv * xv, axis=-1, keepdims=True)
    t = xv * lax.rsqrt(ms + EPS) * g_ref[...]
    tf_ref[...] = t
    logits = jnp.dot(t.astype(jnp.bfloat16), rw_ref[...].astype(jnp.bfloat16),
                     preferred_element_type=jnp.float32)
    m = jnp.max(logits, axis=-1, keepdims=True)
    p = jnp.exp(logits - m)
    probs = p / jnp.sum(p, axis=-1, keepdims=True)
    cols = lax.broadcasted_iota(jnp.int32, probs.shape, 1)
    m1 = jnp.max(probs, axis=-1, keepdims=True)
    idx1 = jnp.min(jnp.where(probs == m1, cols, NUM_EXPERTS),
                   axis=-1, keepdims=True)
    probs2 = jnp.where(cols == idx1, -jnp.inf, probs)
    m2 = jnp.max(probs2, axis=-1, keepdims=True)
    idx2 = jnp.min(jnp.where(probs2 == m2, cols, NUM_EXPERTS),
                   axis=-1, keepdims=True)
    denom = m1 + m2
    g0b = jnp.broadcast_to(m1 / denom, (T, 16))
    g1b = jnp.broadcast_to(m2 / denom, (T, 16))
    gates_ref[...] = jnp.concatenate([g0b, g1b], axis=-1)

    o0 = (cols == idx1).astype(jnp.float32)
    o1 = (cols == idx2).astype(jnp.float32)
    occ = o0 + o1
    rt = lax.broadcasted_iota(jnp.int32, (T, T), 0)
    ct = lax.broadcasted_iota(jnp.int32, (T, T), 1)
    ls = (ct < rt).astype(jnp.bfloat16)
    cume = jnp.dot(ls, occ.astype(jnp.bfloat16),
                   preferred_element_type=jnp.float32)
    counts = jnp.sum(occ, axis=0, keepdims=True)
    ci = counts.astype(jnp.int32)
    pci = ((ci + TM - 1) // TM) * TM
    pcf = pci.astype(jnp.float32)
    r8 = lax.broadcasted_iota(jnp.int32, (NUM_EXPERTS, NUM_EXPERTS), 0)
    c8 = lax.broadcasted_iota(jnp.int32, (NUM_EXPERTS, NUM_EXPERTS), 1)
    lt8 = (r8 < c8).astype(jnp.float32)
    off = jnp.dot(pcf, lt8, precision=lax.Precision.HIGHEST,
                  preferred_element_type=jnp.float32)
    base = cume + off
    pos0 = jnp.sum(o0 * base, axis=-1, keepdims=True)
    pos1 = jnp.sum(o1 * base, axis=-1, keepdims=True)
    pos_ref[...] = jnp.concatenate([pos0, pos1], axis=-1).astype(jnp.int32)

    bvals = (lax.broadcasted_iota(jnp.int32, (1, NB), 1) * TM).astype(
        jnp.float32)
    acc = jnp.full((1, NB), -1, dtype=jnp.int32)
    for e in range(NUM_EXPERTS):
        off_e = lax.slice(off, (0, e), (1, e + 1))
        acc = acc + (bvals >= off_e).astype(jnp.int32)
    bexp_ref[...] = acc


def _route(xf, gm, rw):
    return pl.pallas_call(
        _route_body,
        in_specs=[
            pl.BlockSpec((T, HIDDEN), lambda: (0, 0)),
            pl.BlockSpec((1, HIDDEN), lambda: (0, 0)),
            pl.BlockSpec((HIDDEN, NUM_EXPERTS), lambda: (0, 0)),
        ],
        out_specs=[
            pl.BlockSpec((T, HIDDEN), lambda: (0, 0)),
            pl.BlockSpec((T, 2), lambda: (0, 0)),
            pl.BlockSpec((T, 32), lambda: (0, 0)),
            pl.BlockSpec((1, NB), lambda: (0, 0)),
        ],
        out_shape=[
            jax.ShapeDtypeStruct((T, HIDDEN), jnp.float32),
            jax.ShapeDtypeStruct((T, 2), jnp.int32),
            jax.ShapeDtypeStruct((T, 32), jnp.float32),
            jax.ShapeDtypeStruct((1, NB), jnp.int32),
        ],
    )(xf, gm, rw)


def _make_dispatch():
    mesh = plsc.VectorSubcoreMesh(core_axis_name="c", subcore_axis_name="s")

    rpr = 32
    nr = APW // rpr

    @functools.partial(
        pl.kernel, mesh=mesh,
        out_type=jax.ShapeDtypeStruct((NBTM, HIDDEN), jnp.float32),
        scratch_types=[
            pltpu.VMEM((2, rpr), jnp.int32),
            pltpu.VMEM((2, rpr), jnp.int32),
            pltpu.VMEM((2, rpr, HIDDEN), jnp.float32),
            pltpu.SemaphoreType.DMA,
            pltpu.SemaphoreType.DMA,
            pltpu.SemaphoreType.DMA,
            pltpu.SemaphoreType.DMA,
        ],
    )
    def dispatch(tf_hbm, pos_hbm, xs_hbm, tokv, posv, rows,
                 sg0, sg1, ss0, ss1):
        wid = lax.axis_index("s") * 2 + lax.axis_index("c")
        lanes = lax.iota(jnp.int32, 16)
        sgs = (sg0, sg1)
        sss = (ss0, ss1)
        scat = {}
        for r in range(nr):
            bb = r & 1
            if r >= 2:
                scat[r - 2].wait()
            pltpu.sync_copy(pos_hbm.at[wid, r], posv.at[bb])
            tok0 = wid * (APW // 2) + r * (rpr // 2)
            tv = tokv.at[bb]
            for j in range(rpr // 16):
                tv[pl.ds(j * 16, 16)] = tok0 + ((lanes + j * 16) >> 1)
            g = pltpu.async_copy(tf_hbm.at[tv], rows.at[bb], sgs[bb])
            g.wait()
            scat[r] = pltpu.async_copy(rows.at[bb], xs_hbm.at[posv.at[bb]],
                                       sss[bb])
        scat[nr - 2].wait()
        scat[nr - 1].wait()

    return dispatch


def _mlp_body(be_ref, xs_ref, wg_ref, wl_ref, bgu_ref, wd_ref, bd_ref,
              ys_ref):
    del be_ref
    xb = xs_ref[...].astype(jnp.bfloat16)
    bgu = bgu_ref[0]
    hg = jnp.dot(xb, wg_ref[0], preferred_element_type=jnp.float32) + bgu[0:1]
    hl = jnp.dot(xb, wl_ref[0], preferred_element_type=jnp.float32) + bgu[1:2]
    hg = jnp.minimum(hg, 7.0)
    hl = jnp.clip(hl, -7.0, 7.0)
    act = hg * jax.nn.sigmoid(1.702 * hg) * (hl + 1.0)
    ys_ref[...] = (jnp.dot(act.astype(jnp.bfloat16), wd_ref[0],
                           preferred_element_type=jnp.float32) + bd_ref[0])


def _mlp(bexp, xs, wg, wl, bgu2, wd, bd3):
    grid_spec = pltpu.PrefetchScalarGridSpec(
        num_scalar_prefetch=1,
        grid=(NB,),
        in_specs=[
            pl.BlockSpec((TM, HIDDEN), lambda b, be: (b, 0)),
            pl.BlockSpec((1, HIDDEN, INTER), lambda b, be: (be[b], 0, 0)),
            pl.BlockSpec((1, HIDDEN, INTER), lambda b, be: (be[b], 0, 0)),
            pl.BlockSpec((1, 2, INTER), lambda b, be: (be[b], 0, 0)),
            pl.BlockSpec((1, INTER, HIDDEN), lambda b, be: (be[b], 0, 0)),
            pl.BlockSpec((1, 1, HIDDEN), lambda b, be: (be[b], 0, 0)),
        ],
        out_specs=pl.BlockSpec((TM, HIDDEN), lambda b, be: (b, 0)),
    )
    return pl.pallas_call(
        _mlp_body,
        grid_spec=grid_spec,
        out_shape=jax.ShapeDtypeStruct((NBTM, HIDDEN), jnp.float32),
        compiler_params=pltpu.CompilerParams(
            dimension_semantics=("arbitrary",),
        ),
    )(bexp, xs, wg, wl, bgu2, wd, bd3)


def _make_combine():
    mesh = plsc.VectorSubcoreMesh(core_axis_name="c", subcore_axis_name="s")

    nch = TPW // CH

    @functools.partial(
        pl.kernel, mesh=mesh,
        out_type=jax.ShapeDtypeStruct((T, HIDDEN), jnp.float32),
        scratch_types=[
            pltpu.VMEM((2, 2 * CH), jnp.int32),
            pltpu.VMEM((2, CH, 32), jnp.float32),
            pltpu.VMEM((2, 2 * CH, HIDDEN), jnp.float32),
            pltpu.VMEM((2, CH, HIDDEN), jnp.float32),
            pltpu.SemaphoreType.DMA,
            pltpu.SemaphoreType.DMA,
            pltpu.SemaphoreType.DMA,
            pltpu.SemaphoreType.DMA,
        ],
    )
    def combine(ys_hbm, pos_hbm, gates_hbm, out_hbm, pv, gm, rows, ov,
                sg0, sg1, sw0, sw1):
        wid = lax.axis_index("s") * 2 + lax.axis_index("c")
        sgs = (sg0, sg1)
        sws = (sw0, sw1)
        gath = {}
        writes = {}

        def stage_in(c):
            bb = c & 1
            tok0 = wid * TPW + c * CH
            pltpu.sync_copy(pos_hbm.at[pl.ds(2 * tok0, 2 * CH)], pv.at[bb])
            pltpu.sync_copy(gates_hbm.at[pl.ds(tok0, CH)], gm.at[bb])
            gath[c] = pltpu.async_copy(ys_hbm.at[pv.at[bb]], rows.at[bb],
                                       sgs[bb])

        stage_in(0)
        for c in range(nch):
            bb = c & 1
            if c + 1 < nch:
                stage_in(c + 1)
            gath[c].wait()
            if c >= 2:
                writes[c - 2].wait()
            gmb = gm.at[bb]
            rb = rows.at[bb]
            ob = ov.at[bb]

            def body(j, carry):
                g0 = gmb[j, pl.ds(0, 16)]
                g1 = gmb[j, pl.ds(16, 16)]
                for u in range(HIDDEN // 16):
                    s = pl.ds(u * 16, 16)
                    ob[j, s] = g0 * rb[2 * j, s] + g1 * rb[2 * j + 1, s]
                return carry

            lax.fori_loop(0, CH, body, 0)
            tok0 = wid * TPW + c * CH
            writes[c] = pltpu.async_copy(ov.at[bb],
                                         out_hbm.at[pl.ds(tok0, CH)],
                                         sws[bb])
        writes[nch - 2].wait()
        writes[nch - 1].wait()

    return combine


def kernel(x, norm_g, router_w, W_gu, b_gu, W_d, b_d):
    b, s, h = x.shape
    xf = x.reshape(T, h)
    gm = norm_g.reshape(1, h)
    wgu4 = W_gu.reshape(NUM_EXPERTS, h, INTER, 2)
    W_g = wgu4[..., 0].astype(jnp.bfloat16)
    W_l = wgu4[..., 1].astype(jnp.bfloat16)
    bgu2 = jnp.swapaxes(b_gu.reshape(NUM_EXPERTS, INTER, 2), 1, 2)
    W_d_bf = W_d.astype(jnp.bfloat16)
    bd3 = b_d.reshape(NUM_EXPERTS, 1, h)

    tf, pos2, gates32, bexp = _route(xf, gm, router_w)

    dispatch = _make_dispatch()
    xs = dispatch(tf, pos2.reshape(NW, APW // 32, 32))

    ys = _mlp(bexp.reshape(NB), xs, W_g, W_l, bgu2, W_d_bf, bd3)

    combine = _make_combine()
    out = combine(ys, pos2.reshape(A), gates32)
    return out.reshape(b, s, h)


# speedup vs baseline: 1.6012x; 1.5832x over previous
import functools
import jax
import jax.numpy as jnp
from jax import lax
from jax.experimental import pallas as pl
from jax.experimental.pallas import tpu as pltpu
from jax.experimental.pallas import tpu_sc as plsc

NUM_EXPERTS = 8
HIDDEN = 1024
INTER = 2048
EPS = 1e-6
T = 2048
A = 2 * T
TM = 256
NB = 24
NBTM = NB * TM
NW = 32
APW = A // NW
TPW = T // NW
CH = 16


def _route_body(x_ref, g_ref, rw_ref, tf_ref, pos_ref, gates_ref, bexp_ref):
    xv = x_ref[...]
    ms = jnp.mean(xv * xv, axis=-1, keepdims=True)
    t = xv * lax.rsqrt(ms + EPS) * g_ref[...]
    tf_ref[...] = t
    logits = jnp.dot(t.astype(jnp.bfloat16), rw_ref[...].astype(jnp.bfloat16),
                     preferred_element_type=jnp.float32)
    m = jnp.max(logits, axis=-1, keepdims=True)
    p = jnp.exp(logits - m)
    probs = p / jnp.sum(p, axis=-1, keepdims=True)
    cols = lax.broadcasted_iota(jnp.int32, probs.shape, 1)
    m1 = jnp.max(probs, axis=-1, keepdims=True)
    idx1 = jnp.min(jnp.where(probs == m1, cols, NUM_EXPERTS),
                   axis=-1, keepdims=True)
    probs2 = jnp.where(cols == idx1, -jnp.inf, probs)
    m2 = jnp.max(probs2, axis=-1, keepdims=True)
    idx2 = jnp.min(jnp.where(probs2 == m2, cols, NUM_EXPERTS),
                   axis=-1, keepdims=True)
    denom = m1 + m2
    g0b = jnp.broadcast_to(m1 / denom, (T, 16))
    g1b = jnp.broadcast_to(m2 / denom, (T, 16))
    gates_ref[...] = jnp.concatenate([g0b, g1b], axis=-1)

    o0 = (cols == idx1).astype(jnp.float32)
    o1 = (cols == idx2).astype(jnp.float32)
    occ = o0 + o1
    rt = lax.broadcasted_iota(jnp.int32, (T, T), 0)
    ct = lax.broadcasted_iota(jnp.int32, (T, T), 1)
    ls = (ct < rt).astype(jnp.bfloat16)
    cume = jnp.dot(ls, occ.astype(jnp.bfloat16),
                   preferred_element_type=jnp.float32)
    counts = jnp.sum(occ, axis=0, keepdims=True)
    ci = counts.astype(jnp.int32)
    pci = ((ci + TM - 1) // TM) * TM
    pcf = pci.astype(jnp.float32)
    r8 = lax.broadcasted_iota(jnp.int32, (NUM_EXPERTS, NUM_EXPERTS), 0)
    c8 = lax.broadcasted_iota(jnp.int32, (NUM_EXPERTS, NUM_EXPERTS), 1)
    lt8 = (r8 < c8).astype(jnp.float32)
    off = jnp.dot(pcf, lt8, precision=lax.Precision.HIGHEST,
                  preferred_element_type=jnp.float32)
    base = cume + off
    pos0 = jnp.sum(o0 * base, axis=-1, keepdims=True)
    pos1 = jnp.sum(o1 * base, axis=-1, keepdims=True)
    pos_ref[...] = jnp.concatenate([pos0, pos1], axis=-1).astype(jnp.int32)

    bvals = (lax.broadcasted_iota(jnp.int32, (1, NB), 1) * TM).astype(
        jnp.float32)
    acc = jnp.full((1, NB), -1, dtype=jnp.int32)
    for e in range(NUM_EXPERTS):
        off_e = lax.slice(off, (0, e), (1, e + 1))
        acc = acc + (bvals >= off_e).astype(jnp.int32)
    bexp_ref[...] = acc


def _route(xf, gm, rw):
    return pl.pallas_call(
        _route_body,
        in_specs=[
            pl.BlockSpec((T, HIDDEN), lambda: (0, 0)),
            pl.BlockSpec((1, HIDDEN), lambda: (0, 0)),
            pl.BlockSpec((HIDDEN, NUM_EXPERTS), lambda: (0, 0)),
        ],
        out_specs=[
            pl.BlockSpec((T, HIDDEN), lambda: (0, 0)),
            pl.BlockSpec((T, 2), lambda: (0, 0)),
            pl.BlockSpec((T, 32), lambda: (0, 0)),
            pl.BlockSpec((1, NB), lambda: (0, 0)),
        ],
        out_shape=[
            jax.ShapeDtypeStruct((T, HIDDEN), jnp.float32),
            jax.ShapeDtypeStruct((T, 2), jnp.int32),
            jax.ShapeDtypeStruct((T, 32), jnp.float32),
            jax.ShapeDtypeStruct((1, NB), jnp.int32),
        ],
    )(xf, gm, rw)


def _make_dispatch():
    mesh = plsc.VectorSubcoreMesh(core_axis_name="c", subcore_axis_name="s")

    rpr = 32
    nr = APW // rpr

    @functools.partial(
        pl.kernel, mesh=mesh,
        out_type=jax.ShapeDtypeStruct((NBTM, HIDDEN), jnp.float32),
        scratch_types=[
            pltpu.VMEM((2, rpr), jnp.int32),
            pltpu.VMEM((2, rpr), jnp.int32),
            pltpu.VMEM((2, rpr, HIDDEN), jnp.float32),
            pltpu.SemaphoreType.DMA,
            pltpu.SemaphoreType.DMA,
            pltpu.SemaphoreType.DMA,
            pltpu.SemaphoreType.DMA,
        ],
    )
    def dispatch(tf_hbm, pos_hbm, xs_hbm, tokv, posv, rows,
                 sg0, sg1, ss0, ss1):
        wid = lax.axis_index("s") * 2 + lax.axis_index("c")
        lanes = lax.iota(jnp.int32, 16)
        sgs = (sg0, sg1)
        sss = (ss0, ss1)
        scat = {}
        for r in range(nr):
            bb = r & 1
            if r >= 2:
                scat[r - 2].wait()
            pltpu.sync_copy(pos_hbm.at[wid, r], posv.at[bb])
            tok0 = wid * (APW // 2) + r * (rpr // 2)
            tv = tokv.at[bb]
            for j in range(rpr // 16):
                tv[pl.ds(j * 16, 16)] = tok0 + ((lanes + j * 16) >> 1)
            g = pltpu.async_copy(tf_hbm.at[tv], rows.at[bb], sgs[bb])
            g.wait()
            scat[r] = pltpu.async_copy(rows.at[bb], xs_hbm.at[posv.at[bb]],
                                       sss[bb])
        scat[nr - 2].wait()
        scat[nr - 1].wait()

    return dispatch


def _mlp_body(be_ref, xs_ref, wgu_ref, bgu_ref, wd_ref, bd_ref, ys_ref):
    del be_ref
    xb = xs_ref[...].astype(jnp.bfloat16)
    h1 = jnp.dot(xb, wgu_ref[0], preferred_element_type=jnp.float32)
    h1b = h1.astype(jnp.bfloat16)
    ri = lax.broadcasted_iota(jnp.int32, (256, 128), 0)
    ci = lax.broadcasted_iota(jnp.int32, (256, 128), 1)
    se = (ri == 2 * ci).astype(jnp.bfloat16)
    so = (ri == 2 * ci + 1).astype(jnp.bfloat16)
    hg_parts = []
    hl_parts = []
    for g in range(2 * INTER // 256):
        h1g = h1b[:, 256 * g:256 * (g + 1)]
        hg_parts.append(jnp.dot(h1g, se, preferred_element_type=jnp.float32))
        hl_parts.append(jnp.dot(h1g, so, preferred_element_type=jnp.float32))
    bgu = bgu_ref[0]
    hg = jnp.concatenate(hg_parts, axis=1) + bgu[0:1]
    hl = jnp.concatenate(hl_parts, axis=1) + bgu[1:2]
    hg = jnp.minimum(hg, 7.0)
    hl = jnp.clip(hl, -7.0, 7.0)
    act = hg * jax.nn.sigmoid(1.702 * hg) * (hl + 1.0)
    ys_ref[...] = (jnp.dot(act.astype(jnp.bfloat16), wd_ref[0],
                           preferred_element_type=jnp.float32) + bd_ref[0])


def _mlp(bexp, xs, wgu_bf, bgu2, wd, bd3):
    grid_spec = pltpu.PrefetchScalarGridSpec(
        num_scalar_prefetch=1,
        grid=(NB,),
        in_specs=[
            pl.BlockSpec((TM, HIDDEN), lambda b, be: (b, 0)),
            pl.BlockSpec((1, HIDDEN, 2 * INTER), lambda b, be: (be[b], 0, 0)),
            pl.BlockSpec((1, 2, INTER), lambda b, be: (be[b], 0, 0)),
            pl.BlockSpec((1, INTER, HIDDEN), lambda b, be: (be[b], 0, 0)),
            pl.BlockSpec((1, 1, HIDDEN), lambda b, be: (be[b], 0, 0)),
        ],
        out_specs=pl.BlockSpec((TM, HIDDEN), lambda b, be: (b, 0)),
    )
    return pl.pallas_call(
        _mlp_body,
        grid_spec=grid_spec,
        out_shape=jax.ShapeDtypeStruct((NBTM, HIDDEN), jnp.float32),
        compiler_params=pltpu.CompilerParams(
            dimension_semantics=("arbitrary",),
        ),
    )(bexp, xs, wgu_bf, bgu2, wd, bd3)


def _make_combine():
    mesh = plsc.VectorSubcoreMesh(core_axis_name="c", subcore_axis_name="s")

    nch = TPW // CH

    @functools.partial(
        pl.kernel, mesh=mesh,
        out_type=jax.ShapeDtypeStruct((T, HIDDEN), jnp.float32),
        scratch_types=[
            pltpu.VMEM((2, 2 * CH), jnp.int32),
            pltpu.VMEM((2, CH, 32), jnp.float32),
            pltpu.VMEM((2, 2 * CH, HIDDEN), jnp.float32),
            pltpu.VMEM((2, CH, HIDDEN), jnp.float32),
            pltpu.SemaphoreType.DMA,
            pltpu.SemaphoreType.DMA,
            pltpu.SemaphoreType.DMA,
            pltpu.SemaphoreType.DMA,
        ],
    )
    def combine(ys_hbm, pos_hbm, gates_hbm, out_hbm, pv, gm, rows, ov,
                sg0, sg1, sw0, sw1):
        wid = lax.axis_index("s") * 2 + lax.axis_index("c")
        sgs = (sg0, sg1)
        sws = (sw0, sw1)
        gath = {}
        writes = {}

        def stage_in(c):
            bb = c & 1
            tok0 = wid * TPW + c * CH
            pltpu.sync_copy(pos_hbm.at[pl.ds(2 * tok0, 2 * CH)], pv.at[bb])
            pltpu.sync_copy(gates_hbm.at[pl.ds(tok0, CH)], gm.at[bb])
            gath[c] = pltpu.async_copy(ys_hbm.at[pv.at[bb]], rows.at[bb],
                                       sgs[bb])

        stage_in(0)
        for c in range(nch):
            bb = c & 1
            if c + 1 < nch:
                stage_in(c + 1)
            gath[c].wait()
            if c >= 2:
                writes[c - 2].wait()
            gmb = gm.at[bb]
            rb = rows.at[bb]
            ob = ov.at[bb]

            def body(j, carry):
                g0 = gmb[j, pl.ds(0, 16)]
                g1 = gmb[j, pl.ds(16, 16)]
                for u in range(HIDDEN // 16):
                    s = pl.ds(u * 16, 16)
                    ob[j, s] = g0 * rb[2 * j, s] + g1 * rb[2 * j + 1, s]
                return carry

            lax.fori_loop(0, CH, body, 0)
            tok0 = wid * TPW + c * CH
            writes[c] = pltpu.async_copy(ov.at[bb],
                                         out_hbm.at[pl.ds(tok0, CH)],
                                         sws[bb])
        writes[nch - 2].wait()
        writes[nch - 1].wait()

    return combine


def kernel(x, norm_g, router_w, W_gu, b_gu, W_d, b_d):
    b, s, h = x.shape
    xf = x.reshape(T, h)
    gm = norm_g.reshape(1, h)
    W_gu_bf = W_gu.astype(jnp.bfloat16)
    bgu2 = jnp.swapaxes(b_gu.reshape(NUM_EXPERTS, INTER, 2), 1, 2)
    W_d_bf = W_d.astype(jnp.bfloat16)
    bd3 = b_d.reshape(NUM_EXPERTS, 1, h)

    tf, pos2, gates32, bexp = _route(xf, gm, router_w)

    dispatch = _make_dispatch()
    xs = dispatch(tf, pos2.reshape(NW, APW // 32, 32))

    ys = _mlp(bexp.reshape(NB), xs, W_gu_bf, bgu2, W_d_bf, bd3)

    combine = _make_combine()
    out = combine(ys, pos2.reshape(A), gates32)
    return out.reshape(b, s, h)
